# baseline (speedup 1.0000x reference)
import functools

import jax
import jax.numpy as jnp
from jax import lax
from jax.experimental import pallas as pl
from jax.experimental.pallas import tpu as pltpu
from jax.experimental.pallas import tpu_sc as plsc

MAXLEN = 200
EMBED = 64
LANES = 16
NC, NS = 2, 16
NW = NC * NS
CHUNK = 100
CPR = MAXLEN // CHUNK


def _sc_embed(x2d, token_table, pos_table):
    n_chunks = x2d.shape[0]
    cpw = n_chunks // NW
    mesh = plsc.VectorSubcoreMesh(core_axis_name="c", subcore_axis_name="s")

    @functools.partial(
        pl.kernel,
        out_type=jax.ShapeDtypeStruct((n_chunks, CHUNK, EMBED), jnp.float32),
        mesh=mesh,
        scratch_types=[
            pltpu.VMEM((cpw, CHUNK), jnp.int32),
            pltpu.VMEM((CHUNK, EMBED), jnp.float32),
            pltpu.VMEM((MAXLEN, EMBED), jnp.float32),
            pltpu.SemaphoreType.DMA,
        ],
        compiler_params=pltpu.CompilerParams(use_tc_tiling_on_sc=False),
    )
    def k(x_hbm, tok_hbm, pos_hbm, out_hbm, idx_v, rows_v, pos_v, sem):
        wid = lax.axis_index("s") * NC + lax.axis_index("c")
        base = wid * cpw
        pltpu.sync_copy(pos_hbm, pos_v)
        pltpu.sync_copy(x_hbm.at[pl.ds(base, cpw)], idx_v)

        def chunk_body(c, carry):
            pltpu.async_copy(tok_hbm.at[idx_v.at[c]], rows_v, sem).wait()
            p0 = (c % CPR) * CHUNK

            def add_body(i, carry2):
                for kk in range(EMBED // LANES):
                    sl = pl.ds(kk * LANES, LANES)
                    rows_v[i, sl] = rows_v[i, sl] + pos_v[p0 + i, sl]
                return carry2

            lax.fori_loop(0, CHUNK, add_body, 0)
            pltpu.sync_copy(rows_v, out_hbm.at[base + c])
            return carry

        lax.fori_loop(0, cpw, chunk_body, 0)

    return k(x2d, token_table, pos_table)


def kernel(x, token_table, pos_table):
    batch, seqlen = x.shape
    x2d = x.reshape(-1, CHUNK).astype(jnp.int32)
    out = _sc_embed(x2d, token_table, pos_table)
    return out.reshape(batch, seqlen, EMBED)


# speedup vs baseline: 1.4705x; 1.4705x over previous
import functools

import jax
import jax.numpy as jnp
from jax import lax
from jax.experimental import pallas as pl
from jax.experimental.pallas import tpu as pltpu
from jax.experimental.pallas import tpu_sc as plsc

MAXLEN = 200
EMBED = 64
LANES = 16
NC, NS = 2, 16
NW = NC * NS
CHUNK = 100
ROWS_PER_STEP = 4


def _sc_embed(x2d, token_table, pos_table):
    n_chunks = x2d.shape[0]
    cpw = n_chunks // NW
    steps = cpw // 2
    mesh = plsc.VectorSubcoreMesh(core_axis_name="c", subcore_axis_name="s")

    @functools.partial(
        pl.kernel,
        out_type=jax.ShapeDtypeStruct((n_chunks, CHUNK, EMBED), jnp.float32),
        mesh=mesh,
        scratch_types=[
            pltpu.VMEM((cpw, CHUNK), jnp.int32),
            pltpu.VMEM((CHUNK, EMBED), jnp.float32),
            pltpu.VMEM((CHUNK, EMBED), jnp.float32),
            pltpu.VMEM((MAXLEN, EMBED), jnp.float32),
            pltpu.SemaphoreType.DMA,
            pltpu.SemaphoreType.DMA,
            pltpu.SemaphoreType.DMA,
            pltpu.SemaphoreType.DMA,
        ],
        compiler_params=pltpu.CompilerParams(use_tc_tiling_on_sc=False),
    )
    def k(x_hbm, tok_hbm, pos_hbm, out_hbm, idx_v, rows0, rows1, pos_v,
          gsem0, gsem1, ssem0, ssem1):
        wid = lax.axis_index("s") * NC + lax.axis_index("c")
        base = wid * cpw
        pltpu.sync_copy(pos_hbm, pos_v)
        pltpu.sync_copy(x_hbm.at[pl.ds(base, cpw)], idx_v)

        def add_pos(rows, p0):
            def body(i, carry):
                i0 = i * ROWS_PER_STEP
                for r in range(ROWS_PER_STEP):
                    for kk in range(EMBED // LANES):
                        sl = pl.ds(kk * LANES, LANES)
                        rows[i0 + r, sl] = rows[i0 + r, sl] + pos_v[p0 + i0 + r, sl]
                return carry
            lax.fori_loop(0, CHUNK // ROWS_PER_STEP, body, 0)

        pltpu.async_copy(tok_hbm.at[idx_v.at[0]], rows0, gsem0)

        def step(g, carry):
            c0 = 2 * g

            @pl.when(g > 0)
            def _():
                pltpu.make_async_copy(rows1, out_hbm.at[base + c0 - 1], ssem1).wait()

            pltpu.async_copy(tok_hbm.at[idx_v.at[c0 + 1]], rows1, gsem1)
            pltpu.make_async_copy(tok_hbm.at[idx_v.at[c0]], rows0, gsem0).wait()
            add_pos(rows0, 0)
            pltpu.async_copy(rows0, out_hbm.at[base + c0], ssem0)

            @pl.when(g < steps - 1)
            def _():
                pltpu.make_async_copy(rows0, out_hbm.at[base + c0], ssem0).wait()
                pltpu.async_copy(tok_hbm.at[idx_v.at[c0 + 2]], rows0, gsem0)

            pltpu.make_async_copy(tok_hbm.at[idx_v.at[c0 + 1]], rows1, gsem1).wait()
            add_pos(rows1, CHUNK)
            pltpu.async_copy(rows1, out_hbm.at[base + c0 + 1], ssem1)
            return carry

        lax.fori_loop(0, steps, step, 0)
        pltpu.make_async_copy(rows0, out_hbm.at[base + cpw - 2], ssem0).wait()
        pltpu.make_async_copy(rows1, out_hbm.at[base + cpw - 1], ssem1).wait()

    return k(x2d, token_table, pos_table)


def kernel(x, token_table, pos_table):
    batch, seqlen = x.shape
    x2d = x.reshape(-1, CHUNK).astype(jnp.int32)
    out = _sc_embed(x2d, token_table, pos_table)
    return out.reshape(batch, seqlen, EMBED)
